# Initial kernel scaffold; baseline (speedup 1.0000x reference)
import jax
import jax.numpy as jnp
from jax.experimental import pallas as pl


def kernel(x, Wr, br, Wn, bn, w1, w2, w3):
    raise NotImplementedError("write your pallas kernel here")


# speedup vs baseline: 1.1161x; 1.1161x over previous
import jax
import jax.numpy as jnp
from jax.experimental import pallas as pl
from jax.experimental.pallas import tpu as pltpu

_T, _D, _H, _E, _K = 64, 1024, 2048, 64, 2
_HC = 1024


def _router_kernel(x_ref, Wr_ref, br_ref, Wn_ref, bn_ref, noise_ref,
                   G_ref, ids_ref, n_ref):
    x = x_ref[...]
    logits = jnp.dot(x, Wr_ref[...], preferred_element_type=jnp.float32) + br_ref[...]
    nl = jnp.dot(x, Wn_ref[...], preferred_element_type=jnp.float32) + bn_ref[...]
    noisy = logits + noise_ref[...] * jax.nn.softplus(nl)

    ecols = jax.lax.broadcasted_iota(jnp.int32, (_T, _E), 1)
    m0 = jnp.max(noisy, axis=1, keepdims=True)
    i0 = jnp.min(jnp.where(noisy == m0, ecols, _E), axis=1, keepdims=True)
    masked = jnp.where(ecols == i0, -jnp.inf, noisy)
    m1 = jnp.max(masked, axis=1, keepdims=True)
    i1 = jnp.min(jnp.where(masked == m1, ecols, _E), axis=1, keepdims=True)
    r = jnp.exp(m1 - m0)
    g0 = 1.0 / (1.0 + r)
    g1 = r / (1.0 + r)
    G_ref[...] = jnp.where(ecols == i0, g0, 0.0) + jnp.where(ecols == i1, g1, 0.0)

    sel = ((ecols == i0) | (ecols == i1)).astype(jnp.int32)
    am = jnp.max(sel, axis=0, keepdims=True)
    esub = jax.lax.broadcasted_iota(jnp.int32, (_E, _E), 0)
    elane = jax.lax.broadcasted_iota(jnp.int32, (_E, _E), 1)
    tri = (esub <= elane).astype(jnp.float32)
    cum = jnp.dot(am.astype(jnp.float32), tri,
                  preferred_element_type=jnp.float32).astype(jnp.int32)
    n = cum[0:1, _E - 1:_E]
    diag = (esub == elane).astype(jnp.int32)
    am_col = jnp.sum(diag * am, axis=1, keepdims=True)
    cum_col = jnp.sum(diag * cum, axis=1, keepdims=True)
    match = (am_col > 0) & (cum_col == elane + 1)
    ids_row = jnp.sum(jnp.where(match, esub, 0), axis=0, keepdims=True)
    last = jnp.max(jnp.where(am_col > 0, esub[:, 0:1], -1), axis=0, keepdims=True)
    jidx = jax.lax.broadcasted_iota(jnp.int32, (1, _E), 1)
    ids_ref[...] = jnp.where(jidx < n, ids_row, last)
    n_ref[...] = n


def _ffn_kernel(ids_ref, n_ref, x_ref, G_ref, w1_ref, w3_ref, w2_ref, out_ref):
    j = pl.program_id(0)
    h = pl.program_id(1)

    @pl.when((j == 0) & (h == 0))
    def _init():
        out_ref[...] = jnp.zeros_like(out_ref)

    @pl.when(j < n_ref[0])
    def _body():
        xb = x_ref[...].astype(jnp.bfloat16)
        hp = jnp.dot(xb, w1_ref[0].astype(jnp.bfloat16),
                     preferred_element_type=jnp.float32)
        gp = jnp.dot(xb, w3_ref[0].astype(jnp.bfloat16),
                     preferred_element_type=jnp.float32)
        s = (hp * jax.nn.sigmoid(hp) * gp).astype(jnp.bfloat16)
        y = jnp.dot(s, w2_ref[0].astype(jnp.bfloat16),
                    preferred_element_type=jnp.float32)
        e = ids_ref[j]
        ecols = jax.lax.broadcasted_iota(jnp.int32, (_T, _E), 1)
        gcol = jnp.sum(jnp.where(ecols == e, G_ref[...], 0.0),
                       axis=1, keepdims=True)
        out_ref[...] += y * gcol


def kernel(x, Wr, br, Wn, bn, w1, w2, w3):
    noise = jax.random.normal(jax.random.key(1234), (_T, _E), dtype=jnp.float32)
    G, ids2d, n2d = pl.pallas_call(
        _router_kernel,
        out_shape=[
            jax.ShapeDtypeStruct((_T, _E), jnp.float32),
            jax.ShapeDtypeStruct((1, _E), jnp.int32),
            jax.ShapeDtypeStruct((1, 1), jnp.int32),
        ],
    )(x, Wr, br.reshape(1, _E), Wn, bn.reshape(1, _E), noise)
    ids = ids2d.reshape(_E)
    n = n2d.reshape(1)

    grid = (_E, _H // _HC)
    out = pl.pallas_call(
        _ffn_kernel,
        grid_spec=pltpu.PrefetchScalarGridSpec(
            num_scalar_prefetch=2,
            grid=grid,
            in_specs=[
                pl.BlockSpec((_T, _D), lambda j, h, ids, n: (0, 0)),
                pl.BlockSpec((_T, _E), lambda j, h, ids, n: (0, 0)),
                pl.BlockSpec((1, _D, _HC), lambda j, h, ids, n: (ids[j], 0, h)),
                pl.BlockSpec((1, _D, _HC), lambda j, h, ids, n: (ids[j], 0, h)),
                pl.BlockSpec((1, _HC, _D), lambda j, h, ids, n: (ids[j], h, 0)),
            ],
            out_specs=pl.BlockSpec((_T, _D), lambda j, h, ids, n: (0, 0)),
        ),
        out_shape=jax.ShapeDtypeStruct((_T, _D), jnp.float32),
        compiler_params=pltpu.CompilerParams(
            dimension_semantics=("arbitrary", "arbitrary"),
        ),
    )(ids, n, x, G, w1, w3, w2)
    return out


# speedup vs baseline: 1.1985x; 1.0738x over previous
import jax
import jax.numpy as jnp
from jax.experimental import pallas as pl
from jax.experimental.pallas import tpu as pltpu

_T, _D, _H, _E, _K = 64, 1024, 2048, 64, 2
_HC = 2048


def _router_kernel(x_ref, Wr_ref, br_ref, Wn_ref, bn_ref, noise_ref,
                   G_ref, ids_ref, n_ref):
    x = x_ref[...]
    logits = jnp.dot(x, Wr_ref[...], preferred_element_type=jnp.float32) + br_ref[...]
    nl = jnp.dot(x, Wn_ref[...], preferred_element_type=jnp.float32) + bn_ref[...]
    noisy = logits + noise_ref[...] * jax.nn.softplus(nl)

    ecols = jax.lax.broadcasted_iota(jnp.int32, (_T, _E), 1)
    m0 = jnp.max(noisy, axis=1, keepdims=True)
    i0 = jnp.min(jnp.where(noisy == m0, ecols, _E), axis=1, keepdims=True)
    masked = jnp.where(ecols == i0, -jnp.inf, noisy)
    m1 = jnp.max(masked, axis=1, keepdims=True)
    i1 = jnp.min(jnp.where(masked == m1, ecols, _E), axis=1, keepdims=True)
    r = jnp.exp(m1 - m0)
    g0 = 1.0 / (1.0 + r)
    g1 = r / (1.0 + r)
    G_ref[...] = jnp.where(ecols == i0, g0, 0.0) + jnp.where(ecols == i1, g1, 0.0)

    sel = ((ecols == i0) | (ecols == i1)).astype(jnp.int32)
    am = jnp.max(sel, axis=0, keepdims=True)
    esub = jax.lax.broadcasted_iota(jnp.int32, (_E, _E), 0)
    elane = jax.lax.broadcasted_iota(jnp.int32, (_E, _E), 1)
    tri = (esub <= elane).astype(jnp.float32)
    cum = jnp.dot(am.astype(jnp.float32), tri,
                  preferred_element_type=jnp.float32).astype(jnp.int32)
    n = cum[0:1, _E - 1:_E]
    diag = (esub == elane).astype(jnp.int32)
    am_col = jnp.sum(diag * am, axis=1, keepdims=True)
    cum_col = jnp.sum(diag * cum, axis=1, keepdims=True)
    match = (am_col > 0) & (cum_col == elane + 1)
    ids_row = jnp.sum(jnp.where(match, esub, 0), axis=0, keepdims=True)
    last = jnp.max(jnp.where(am_col > 0, esub[:, 0:1], -1), axis=0, keepdims=True)
    jidx = jax.lax.broadcasted_iota(jnp.int32, (1, _E), 1)
    ids_ref[...] = jnp.where(jidx < n, ids_row, last)
    n_ref[...] = n


def _ffn_kernel(ids_ref, n_ref, x_ref, G_ref, w1_ref, w3_ref, w2_ref, out_ref):
    j = pl.program_id(0)
    h = pl.program_id(1)

    @pl.when((j == 0) & (h == 0))
    def _init():
        out_ref[...] = jnp.zeros_like(out_ref)

    @pl.when(j < n_ref[0])
    def _body():
        xb = x_ref[...].astype(jnp.bfloat16)
        hp = jnp.dot(xb, w1_ref[0].astype(jnp.bfloat16),
                     preferred_element_type=jnp.float32)
        gp = jnp.dot(xb, w3_ref[0].astype(jnp.bfloat16),
                     preferred_element_type=jnp.float32)
        s = (hp * jax.nn.sigmoid(hp) * gp).astype(jnp.bfloat16)
        y = jnp.dot(s, w2_ref[0].astype(jnp.bfloat16),
                    preferred_element_type=jnp.float32)
        e = ids_ref[j]
        ecols = jax.lax.broadcasted_iota(jnp.int32, (_T, _E), 1)
        gcol = jnp.sum(jnp.where(ecols == e, G_ref[...], 0.0),
                       axis=1, keepdims=True)
        out_ref[...] += y * gcol


def kernel(x, Wr, br, Wn, bn, w1, w2, w3):
    noise = jax.random.normal(jax.random.key(1234), (_T, _E), dtype=jnp.float32)
    G, ids2d, n2d = pl.pallas_call(
        _router_kernel,
        out_shape=[
            jax.ShapeDtypeStruct((_T, _E), jnp.float32),
            jax.ShapeDtypeStruct((1, _E), jnp.int32),
            jax.ShapeDtypeStruct((1, 1), jnp.int32),
        ],
    )(x, Wr, br.reshape(1, _E), Wn, bn.reshape(1, _E), noise)
    ids = ids2d.reshape(_E)
    n = n2d.reshape(1)

    grid = (_E, _H // _HC)
    out = pl.pallas_call(
        _ffn_kernel,
        grid_spec=pltpu.PrefetchScalarGridSpec(
            num_scalar_prefetch=2,
            grid=grid,
            in_specs=[
                pl.BlockSpec((_T, _D), lambda j, h, ids, n: (0, 0)),
                pl.BlockSpec((_T, _E), lambda j, h, ids, n: (0, 0)),
                pl.BlockSpec((1, _D, _HC), lambda j, h, ids, n: (ids[j], 0, h)),
                pl.BlockSpec((1, _D, _HC), lambda j, h, ids, n: (ids[j], 0, h)),
                pl.BlockSpec((1, _HC, _D), lambda j, h, ids, n: (ids[j], h, 0)),
            ],
            out_specs=pl.BlockSpec((_T, _D), lambda j, h, ids, n: (0, 0)),
        ),
        out_shape=jax.ShapeDtypeStruct((_T, _D), jnp.float32),
        compiler_params=pltpu.CompilerParams(
            dimension_semantics=("arbitrary", "arbitrary"),
        ),
    )(ids, n, x, G, w1, w3, w2)
    return out
